# baseline (speedup 1.0000x reference)
import functools

import jax
import jax.numpy as jnp
from jax import lax
from jax.experimental import pallas as pl
from jax.experimental.pallas import tpu as pltpu
from jax.experimental.pallas import tpu_sc as plsc

N = 10000
E = 320000
D = 128
G = 128
DH = D // 2

NC = 2
NS = 16
NW = NC * NS
CHUNK = 128
EPW = E // NW
NCHUNK = 80
EPW2 = NCHUNK * CHUNK
NP = 10240
RPT = NP // NS
ZROWS = 128

MBLK = 1000
NBLK = N // MBLK

_MESH = plsc.VectorSubcoreMesh(core_axis_name="c", subcore_axis_name="s")
_SC_PARAMS = pltpu.CompilerParams(use_tc_tiling_on_sc=False)


@functools.partial(
    pl.kernel,
    out_type=jax.ShapeDtypeStruct((NC, NP, 16), jnp.float32),
    mesh=_MESH,
    scratch_types=[
        pltpu.VMEM((NCHUNK, CHUNK), jnp.int32),
        pltpu.VMEM((CHUNK, 16), jnp.float32),
        pltpu.VMEM((ZROWS, 16), jnp.float32),
        pltpu.VMEM_SHARED((NP, 16), jnp.float32),
        pltpu.SemaphoreType.DMA,
    ],
    compiler_params=_SC_PARAMS,
)
def _deg(dst_hbm, out_hbm, dst_v, ones_v, z_v, acc, sem):
    c = lax.axis_index("c")
    s = lax.axis_index("s")
    wid = c * NS + s

    @pl.loop(0, CHUNK)
    def _(r):
        ones_v[r, :] = jnp.ones((16,), jnp.float32)

    @pl.loop(0, ZROWS)
    def _(r):
        z_v[r, :] = jnp.zeros((16,), jnp.float32)

    for k in range(RPT // ZROWS):
        pltpu.sync_copy(z_v, acc.at[pl.ds(s * RPT + k * ZROWS, ZROWS)])
    plsc.subcore_barrier()

    pltpu.sync_copy(dst_hbm.at[wid], dst_v)

    @pl.loop(0, NCHUNK)
    def _(j):
        pltpu.async_copy(ones_v, acc.at[dst_v.at[j]], sem, add=True)

    @pl.loop(0, NCHUNK)
    def _(j):
        pltpu.make_async_copy(ones_v, acc.at[dst_v.at[j]], sem).wait()

    plsc.subcore_barrier()
    pltpu.sync_copy(acc.at[pl.ds(s * RPT, RPT)],
                    out_hbm.at[c, pl.ds(s * RPT, RPT)])


NBUF = 5
GAHEAD = 2
assert NCHUNK % NBUF == 0 and NCHUNK >= 3 * NBUF


@functools.partial(
    pl.kernel,
    out_type=jax.ShapeDtypeStruct((NC, 2, NP, DH), jnp.float32),
    mesh=_MESH,
    scratch_types=[
        pltpu.VMEM((NCHUNK, CHUNK), jnp.int32),
        pltpu.VMEM((NCHUNK, CHUNK), jnp.int32),
    ] + [pltpu.VMEM((CHUNK, DH), jnp.float32) for _ in range(NBUF)]
      + [pltpu.VMEM((ZROWS, DH), jnp.float32),
         pltpu.VMEM_SHARED((NP, DH), jnp.float32)]
      + [pltpu.SemaphoreType.DMA for _ in range(2 * NBUF)],
    compiler_params=_SC_PARAMS,
)
def _edge_pass(y_hbm, src_hbm, dst_hbm, out_hbm,
               src_v, dst_v, b0, b1, b2, b3, b4, z_v, acc,
               g0, g1, g2, g3, g4, s0, s1, s2, s3, s4):
    c = lax.axis_index("c")
    s = lax.axis_index("s")
    wid = c * NS + s
    bufs = (b0, b1, b2, b3, b4)
    gsems = (g0, g1, g2, g3, g4)
    ssems = (s0, s1, s2, s3, s4)

    @pl.loop(0, ZROWS)
    def _(r):
        for cb in range(DH // 16):
            z_v[r, pl.ds(cb * 16, 16)] = jnp.zeros((16,), jnp.float32)

    pltpu.sync_copy(src_hbm.at[wid], src_v)
    pltpu.sync_copy(dst_hbm.at[wid], dst_v)

    for h in range(2):
        tab = y_hbm.at[h]

        def gath(j, b):
            pltpu.async_copy(tab.at[src_v.at[j]], bufs[b], gsems[b])

        def gwait(j, b):
            pltpu.make_async_copy(tab.at[src_v.at[j]], bufs[b],
                                  gsems[b]).wait()

        def scat(j, b):
            pltpu.async_copy(bufs[b], acc.at[dst_v.at[j]], ssems[b],
                             add=True)

        def swait(j, b):
            pltpu.make_async_copy(bufs[b], acc.at[dst_v.at[j]],
                                  ssems[b]).wait()

        def step(j, b, do_swait, do_gath):
            gwait(j, b)
            scat(j, b)
            bn = (b + GAHEAD) % NBUF
            if do_swait:
                swait(j - (NBUF - GAHEAD), bn)
            if do_gath:
                gath(j + GAHEAD, bn)

        for k in range(RPT // ZROWS):
            pltpu.sync_copy(z_v, acc.at[pl.ds(s * RPT + k * ZROWS, ZROWS)])
        plsc.subcore_barrier()

        for j in range(GAHEAD):
            gath(j, j)
        for j in range(NBUF):
            step(j, j, j >= NBUF - GAHEAD, True)

        @pl.loop(1, NCHUNK // NBUF - 1)
        def _(grp):
            j0 = grp * NBUF
            for k in range(NBUF):
                step(j0 + k, k, True, True)

        for j in range(NCHUNK - NBUF, NCHUNK):
            b = j % NBUF
            step(j, b, j < NCHUNK - GAHEAD, j < NCHUNK - GAHEAD)
        for j in range(NCHUNK - NBUF, NCHUNK):
            swait(j, j % NBUF)

        plsc.subcore_barrier()
        pltpu.sync_copy(acc.at[pl.ds(s * RPT, RPT)],
                        out_hbm.at[c, h, pl.ds(s * RPT, RPT)])


def _dinv_blk(deg_ref):
    d = deg_ref[...]
    return lax.rsqrt(1.0 + d[0, :, 0:1] + d[1, :, 0:1])


def _split_cols(v):
    return jnp.stack([v[:, :DH], v[:, DH:]], axis=0)


def _merge(o_ref, y_ref):
    o = o_ref[...]
    y = y_ref[...]
    t = o[0] + o[1] + y
    return jnp.concatenate([t[0], t[1]], axis=1)


def _mm_scale_body(x_ref, w_ref, deg_ref, y_ref):
    xw = jnp.dot(x_ref[...], w_ref[...], preferred_element_type=jnp.float32)
    y_ref[...] = _split_cols(xw * _dinv_blk(deg_ref))


def _mm_scale(x, w, degp):
    return pl.pallas_call(
        _mm_scale_body,
        grid=(NBLK,),
        in_specs=[
            pl.BlockSpec((MBLK, D), lambda i: (i, 0)),
            pl.BlockSpec((D, D), lambda i: (0, 0)),
            pl.BlockSpec((NC, MBLK, 16), lambda i: (0, i, 0)),
        ],
        out_specs=pl.BlockSpec((2, MBLK, DH), lambda i: (0, i, 0)),
        out_shape=jax.ShapeDtypeStruct((2, N, DH), jnp.float32),
    )(x, w, degp)


def _pool_update(i, g_ref, bt_ref, z):
    @pl.when(i == 0)
    def _():
        g_ref[...] = jnp.zeros_like(g_ref)

    bb = bt_ref[0, 0, :]
    gi = lax.broadcasted_iota(jnp.int32, (G, MBLK), 0)
    m = (gi == bb[None, :]).astype(jnp.float32)
    g_ref[...] += jnp.dot(m, z, preferred_element_type=jnp.float32)


def _combine_mm_body(o_ref, y_ref, deg_ref, b_ref, a_ref, w_ref,
                     z_ref, y2_ref):
    dinv = _dinv_blk(deg_ref)
    u = _merge(o_ref, y_ref) * dinv + b_ref[...]
    z = jnp.where(u >= 0, u, a_ref[...] * u)
    z_ref[...] = z
    y2 = jnp.dot(z, w_ref[...], preferred_element_type=jnp.float32) * dinv
    y2_ref[...] = _split_cols(y2)


def _combine_mm(o, y, degp, b, a, w):
    return pl.pallas_call(
        _combine_mm_body,
        grid=(NBLK,),
        in_specs=[
            pl.BlockSpec((NC, 2, MBLK, DH), lambda i: (0, 0, i, 0)),
            pl.BlockSpec((2, MBLK, DH), lambda i: (0, i, 0)),
            pl.BlockSpec((NC, MBLK, 16), lambda i: (0, i, 0)),
            pl.BlockSpec((1, D), lambda i: (0, 0)),
            pl.BlockSpec((1, D), lambda i: (0, 0)),
            pl.BlockSpec((D, D), lambda i: (0, 0)),
        ],
        out_specs=[
            pl.BlockSpec((MBLK, D), lambda i: (i, 0)),
            pl.BlockSpec((2, MBLK, DH), lambda i: (0, i, 0)),
        ],
        out_shape=[
            jax.ShapeDtypeStruct((N, D), jnp.float32),
            jax.ShapeDtypeStruct((2, N, DH), jnp.float32),
        ],
    )(o, y, degp, b, a, w)


def _pool_body(z_ref, bt_ref, g_ref):
    _pool_update(pl.program_id(0), g_ref, bt_ref, z_ref[...])


def _pool(z, batch3):
    return pl.pallas_call(
        _pool_body,
        grid=(NBLK,),
        in_specs=[
            pl.BlockSpec((MBLK, D), lambda i: (i, 0)),
            pl.BlockSpec((1, 1, MBLK), lambda i: (i, 0, 0)),
        ],
        out_specs=pl.BlockSpec((G, D), lambda i: (0, 0)),
        out_shape=jax.ShapeDtypeStruct((G, D), jnp.float32),
    )(z, batch3)


def _combine_pool_body(o_ref, y_ref, deg_ref, b_ref, a_ref, bt_ref, g1_ref,
                       z_ref, g_ref):
    i = pl.program_id(0)
    dinv = _dinv_blk(deg_ref)
    u = _merge(o_ref, y_ref) * dinv + b_ref[...]
    z = jnp.where(u >= 0, u, a_ref[...] * u)
    z_ref[...] = z

    @pl.when(i == 0)
    def _():
        g_ref[:, :D] = g1_ref[...]
        g_ref[:, D:] = jnp.zeros_like(g1_ref)

    bb = bt_ref[0, 0, :]
    gi = lax.broadcasted_iota(jnp.int32, (G, MBLK), 0)
    m = (gi == bb[None, :]).astype(jnp.float32)
    g_ref[:, D:] += jnp.dot(m, z, preferred_element_type=jnp.float32)


def _combine_pool(o, y, degp, b, a, batch3, g1):
    return pl.pallas_call(
        _combine_pool_body,
        grid=(NBLK,),
        in_specs=[
            pl.BlockSpec((NC, 2, MBLK, DH), lambda i: (0, 0, i, 0)),
            pl.BlockSpec((2, MBLK, DH), lambda i: (0, i, 0)),
            pl.BlockSpec((NC, MBLK, 16), lambda i: (0, i, 0)),
            pl.BlockSpec((1, D), lambda i: (0, 0)),
            pl.BlockSpec((1, D), lambda i: (0, 0)),
            pl.BlockSpec((1, 1, MBLK), lambda i: (i, 0, 0)),
            pl.BlockSpec((G, D), lambda i: (0, 0)),
        ],
        out_specs=[
            pl.BlockSpec((MBLK, D), lambda i: (i, 0)),
            pl.BlockSpec((G, 2 * D), lambda i: (0, 0)),
        ],
        out_shape=[
            jax.ShapeDtypeStruct((N, D), jnp.float32),
            jax.ShapeDtypeStruct((G, 2 * D), jnp.float32),
        ],
    )(o, y, degp, b, a, batch3, g1)


@jax.jit
def kernel(x, edge_index, batch, W1, b1, a1, W2, b2, a2):
    srcr = edge_index[0].reshape(NW, EPW)
    dstr = edge_index[1].reshape(NW, EPW)
    pad_src = jnp.zeros((NW, EPW2 - EPW), jnp.int32)
    pad_dst = jnp.broadcast_to(
        (N + jnp.arange(NW, dtype=jnp.int32))[:, None], (NW, EPW2 - EPW))
    src3 = jnp.concatenate([srcr, pad_src], axis=1).reshape(NW, NCHUNK, CHUNK)
    dst3 = jnp.concatenate([dstr, pad_dst], axis=1).reshape(NW, NCHUNK, CHUNK)
    batch3 = batch.reshape(NBLK, 1, MBLK)
    b1r = b1.reshape(1, D)
    a1r = a1.reshape(1, D)
    b2r = b2.reshape(1, D)
    a2r = a2.reshape(1, D)

    degp = _deg(dst3)
    y1 = _mm_scale(x, W1, degp)
    o1 = _edge_pass(y1, src3, dst3)
    z1, y2 = _combine_mm(o1, y1, degp, b1r, a1r, W2)
    o2 = _edge_pass(y2, src3, dst3)
    g1 = _pool(z1, batch3)
    z2, g = _combine_pool(o2, y2, degp, b2r, a2r, batch3, g1)
    return (z2, g)


# speedup vs baseline: 2.2619x; 2.2619x over previous
import functools

import jax
import jax.numpy as jnp
from jax import lax
from jax.experimental import pallas as pl
from jax.experimental.pallas import tpu as pltpu
from jax.experimental.pallas import tpu_sc as plsc

N = 10000
E = 320000
D = 128
G = 128
DH = D // 2

NC = 2
NS = 16
NW = NC * NS
CHUNK = 80
EPW = E // NW
NCHUNK = EPW // CHUNK
NP = 10240
RPT = NP // NS
ZROWS = 128

MBLK = 1000
NBLK = N // MBLK

_MESH = plsc.VectorSubcoreMesh(core_axis_name="c", subcore_axis_name="s")
_SC_PARAMS = pltpu.CompilerParams(use_tc_tiling_on_sc=False)


@functools.partial(
    pl.kernel,
    out_type=jax.ShapeDtypeStruct((NC, NP, 16), jnp.float32),
    mesh=_MESH,
    scratch_types=[
        pltpu.VMEM((NCHUNK, CHUNK), jnp.int32),
        pltpu.VMEM((CHUNK, 16), jnp.float32),
        pltpu.VMEM((ZROWS, 16), jnp.float32),
        pltpu.VMEM_SHARED((NP, 16), jnp.float32),
        pltpu.SemaphoreType.DMA,
    ],
    compiler_params=_SC_PARAMS,
)
def _deg(dst_hbm, out_hbm, dst_v, ones_v, z_v, acc, sem):
    c = lax.axis_index("c")
    s = lax.axis_index("s")
    wid = c * NS + s

    @pl.loop(0, CHUNK)
    def _(r):
        ones_v[r, :] = jnp.ones((16,), jnp.float32)

    @pl.loop(0, ZROWS)
    def _(r):
        z_v[r, :] = jnp.zeros((16,), jnp.float32)

    for k in range(RPT // ZROWS):
        pltpu.sync_copy(z_v, acc.at[pl.ds(s * RPT + k * ZROWS, ZROWS)])
    plsc.subcore_barrier()

    pltpu.sync_copy(dst_hbm.at[wid], dst_v)

    @pl.loop(0, NCHUNK)
    def _(j):
        pltpu.async_copy(ones_v, acc.at[dst_v.at[j]], sem, add=True)

    @pl.loop(0, NCHUNK)
    def _(j):
        pltpu.make_async_copy(ones_v, acc.at[dst_v.at[j]], sem).wait()

    plsc.subcore_barrier()
    pltpu.sync_copy(acc.at[pl.ds(s * RPT, RPT)],
                    out_hbm.at[c, pl.ds(s * RPT, RPT)])


NBUF = 5
GAHEAD = 2
assert NCHUNK % NBUF == 0 and NCHUNK >= 3 * NBUF


@functools.partial(
    pl.kernel,
    out_type=jax.ShapeDtypeStruct((NC, 2, NP, DH), jnp.float32),
    mesh=_MESH,
    scratch_types=[
        pltpu.VMEM((NCHUNK, CHUNK), jnp.int32),
        pltpu.VMEM((NCHUNK, CHUNK), jnp.int32),
    ] + [pltpu.VMEM((CHUNK, DH), jnp.float32) for _ in range(NBUF)]
      + [pltpu.VMEM((ZROWS, DH), jnp.float32),
         pltpu.VMEM_SHARED((NP, DH), jnp.float32)]
      + [pltpu.SemaphoreType.DMA for _ in range(2 * NBUF)],
    compiler_params=_SC_PARAMS,
)
def _edge_pass(y_hbm, src_hbm, dst_hbm, out_hbm,
               src_v, dst_v, b0, b1, b2, b3, b4, z_v, acc,
               g0, g1, g2, g3, g4, s0, s1, s2, s3, s4):
    c = lax.axis_index("c")
    s = lax.axis_index("s")
    wid = c * NS + s
    bufs = (b0, b1, b2, b3, b4)
    gsems = (g0, g1, g2, g3, g4)
    ssems = (s0, s1, s2, s3, s4)

    @pl.loop(0, ZROWS)
    def _(r):
        for cb in range(DH // 16):
            z_v[r, pl.ds(cb * 16, 16)] = jnp.zeros((16,), jnp.float32)

    pltpu.sync_copy(src_hbm.at[wid], src_v)
    pltpu.sync_copy(dst_hbm.at[wid], dst_v)

    for h in range(2):
        tab = y_hbm.at[h]

        def gath(j, b):
            pltpu.async_copy(tab.at[src_v.at[j]], bufs[b], gsems[b])

        def gwait(j, b):
            pltpu.make_async_copy(tab.at[src_v.at[j]], bufs[b],
                                  gsems[b]).wait()

        def scat(j, b):
            pltpu.async_copy(bufs[b], acc.at[dst_v.at[j]], ssems[b],
                             add=True)

        def swait(j, b):
            pltpu.make_async_copy(bufs[b], acc.at[dst_v.at[j]],
                                  ssems[b]).wait()

        def step(j, b, do_swait, do_gath):
            gwait(j, b)
            scat(j, b)
            bn = (b + GAHEAD) % NBUF
            if do_swait:
                swait(j - (NBUF - GAHEAD), bn)
            if do_gath:
                gath(j + GAHEAD, bn)

        for k in range(RPT // ZROWS):
            pltpu.sync_copy(z_v, acc.at[pl.ds(s * RPT + k * ZROWS, ZROWS)])
        plsc.subcore_barrier()

        for j in range(GAHEAD):
            gath(j, j)
        for j in range(NBUF):
            step(j, j, j >= NBUF - GAHEAD, True)

        @pl.loop(1, NCHUNK // NBUF - 1)
        def _(grp):
            j0 = grp * NBUF
            for k in range(NBUF):
                step(j0 + k, k, True, True)

        for j in range(NCHUNK - NBUF, NCHUNK):
            b = j % NBUF
            step(j, b, j < NCHUNK - GAHEAD, j < NCHUNK - GAHEAD)
        for j in range(NCHUNK - NBUF, NCHUNK):
            swait(j, j % NBUF)

        plsc.subcore_barrier()
        pltpu.sync_copy(acc.at[pl.ds(s * RPT, RPT)],
                        out_hbm.at[c, h, pl.ds(s * RPT, RPT)])


def _dinv_blk(deg_ref):
    d = deg_ref[...]
    return lax.rsqrt(1.0 + d[0, :, 0:1] + d[1, :, 0:1])


def _split_cols(v):
    return jnp.stack([v[:, :DH], v[:, DH:]], axis=0)


def _merge(o_ref, y_ref):
    o = o_ref[...]
    y = y_ref[...]
    t = o[0] + o[1] + y
    return jnp.concatenate([t[0], t[1]], axis=1)


def _mm_scale_body(x_ref, w_ref, deg_ref, y_ref):
    xw = jnp.dot(x_ref[...], w_ref[...], preferred_element_type=jnp.float32)
    y_ref[...] = _split_cols(xw * _dinv_blk(deg_ref))


def _mm_scale(x, w, degp):
    return pl.pallas_call(
        _mm_scale_body,
        grid=(NBLK,),
        in_specs=[
            pl.BlockSpec((MBLK, D), lambda i: (i, 0)),
            pl.BlockSpec((D, D), lambda i: (0, 0)),
            pl.BlockSpec((NC, MBLK, 16), lambda i: (0, i, 0)),
        ],
        out_specs=pl.BlockSpec((2, MBLK, DH), lambda i: (0, i, 0)),
        out_shape=jax.ShapeDtypeStruct((2, N, DH), jnp.float32),
    )(x, w, degp)


def _pool_update(i, g_ref, bt_ref, z):
    @pl.when(i == 0)
    def _():
        g_ref[...] = jnp.zeros_like(g_ref)

    bb = bt_ref[0, 0, :]
    gi = lax.broadcasted_iota(jnp.int32, (G, MBLK), 0)
    m = (gi == bb[None, :]).astype(jnp.float32)
    g_ref[...] += jnp.dot(m, z, preferred_element_type=jnp.float32)


def _combine_mm_body(o_ref, y_ref, deg_ref, b_ref, a_ref, w_ref,
                     z_ref, y2_ref):
    dinv = _dinv_blk(deg_ref)
    u = _merge(o_ref, y_ref) * dinv + b_ref[...]
    z = jnp.where(u >= 0, u, a_ref[...] * u)
    z_ref[...] = z
    y2 = jnp.dot(z, w_ref[...], preferred_element_type=jnp.float32) * dinv
    y2_ref[...] = _split_cols(y2)


def _combine_mm(o, y, degp, b, a, w):
    return pl.pallas_call(
        _combine_mm_body,
        grid=(NBLK,),
        in_specs=[
            pl.BlockSpec((NC, 2, MBLK, DH), lambda i: (0, 0, i, 0)),
            pl.BlockSpec((2, MBLK, DH), lambda i: (0, i, 0)),
            pl.BlockSpec((NC, MBLK, 16), lambda i: (0, i, 0)),
            pl.BlockSpec((1, D), lambda i: (0, 0)),
            pl.BlockSpec((1, D), lambda i: (0, 0)),
            pl.BlockSpec((D, D), lambda i: (0, 0)),
        ],
        out_specs=[
            pl.BlockSpec((MBLK, D), lambda i: (i, 0)),
            pl.BlockSpec((2, MBLK, DH), lambda i: (0, i, 0)),
        ],
        out_shape=[
            jax.ShapeDtypeStruct((N, D), jnp.float32),
            jax.ShapeDtypeStruct((2, N, DH), jnp.float32),
        ],
    )(o, y, degp, b, a, w)


def _pool_body(z_ref, bt_ref, g_ref):
    _pool_update(pl.program_id(0), g_ref, bt_ref, z_ref[...])


def _pool(z, batch3):
    return pl.pallas_call(
        _pool_body,
        grid=(NBLK,),
        in_specs=[
            pl.BlockSpec((MBLK, D), lambda i: (i, 0)),
            pl.BlockSpec((1, 1, MBLK), lambda i: (i, 0, 0)),
        ],
        out_specs=pl.BlockSpec((G, D), lambda i: (0, 0)),
        out_shape=jax.ShapeDtypeStruct((G, D), jnp.float32),
    )(z, batch3)


def _combine_pool_body(o_ref, y_ref, deg_ref, b_ref, a_ref, bt_ref, g1_ref,
                       z_ref, g_ref):
    i = pl.program_id(0)
    dinv = _dinv_blk(deg_ref)
    u = _merge(o_ref, y_ref) * dinv + b_ref[...]
    z = jnp.where(u >= 0, u, a_ref[...] * u)
    z_ref[...] = z

    @pl.when(i == 0)
    def _():
        g_ref[:, :D] = g1_ref[...]
        g_ref[:, D:] = jnp.zeros_like(g1_ref)

    bb = bt_ref[0, 0, :]
    gi = lax.broadcasted_iota(jnp.int32, (G, MBLK), 0)
    m = (gi == bb[None, :]).astype(jnp.float32)
    g_ref[:, D:] += jnp.dot(m, z, preferred_element_type=jnp.float32)


def _combine_pool(o, y, degp, b, a, batch3, g1):
    return pl.pallas_call(
        _combine_pool_body,
        grid=(NBLK,),
        in_specs=[
            pl.BlockSpec((NC, 2, MBLK, DH), lambda i: (0, 0, i, 0)),
            pl.BlockSpec((2, MBLK, DH), lambda i: (0, i, 0)),
            pl.BlockSpec((NC, MBLK, 16), lambda i: (0, i, 0)),
            pl.BlockSpec((1, D), lambda i: (0, 0)),
            pl.BlockSpec((1, D), lambda i: (0, 0)),
            pl.BlockSpec((1, 1, MBLK), lambda i: (i, 0, 0)),
            pl.BlockSpec((G, D), lambda i: (0, 0)),
        ],
        out_specs=[
            pl.BlockSpec((MBLK, D), lambda i: (i, 0)),
            pl.BlockSpec((G, 2 * D), lambda i: (0, 0)),
        ],
        out_shape=[
            jax.ShapeDtypeStruct((N, D), jnp.float32),
            jax.ShapeDtypeStruct((G, 2 * D), jnp.float32),
        ],
    )(o, y, degp, b, a, batch3, g1)


@jax.jit
def kernel(x, edge_index, batch, W1, b1, a1, W2, b2, a2):
    src3 = edge_index[0].reshape(NW, NCHUNK, CHUNK)
    dst3 = edge_index[1].reshape(NW, NCHUNK, CHUNK)
    batch3 = batch.reshape(NBLK, 1, MBLK)
    b1r = b1.reshape(1, D)
    a1r = a1.reshape(1, D)
    b2r = b2.reshape(1, D)
    a2r = a2.reshape(1, D)

    degp = _deg(dst3)
    y1 = _mm_scale(x, W1, degp)
    o1 = _edge_pass(y1, src3, dst3)
    z1, y2 = _combine_mm(o1, y1, degp, b1r, a1r, W2)
    o2 = _edge_pass(y2, src3, dst3)
    g1 = _pool(z1, batch3)
    z2, g = _combine_pool(o2, y2, degp, b2r, a2r, batch3, g1)
    return (z2, g)


# speedup vs baseline: 3.0196x; 1.3350x over previous
import functools

import jax
import jax.numpy as jnp
from jax import lax
from jax.experimental import pallas as pl
from jax.experimental.pallas import tpu as pltpu
from jax.experimental.pallas import tpu_sc as plsc

N = 10000
E = 320000
D = 128
G = 128
DH = D // 2

NC = 2
NS = 16
NW = NC * NS
CHUNK = 128
ROWS = E // CHUNK
NCHB = ROWS // NW
XTRA = ROWS - NCHB * NW
IDXR = NCHB + 1
NP = 10240
RPT = NP // NS
ZROWS = 128

MBLK = 2000
NBLK = N // MBLK

_MESH = plsc.VectorSubcoreMesh(core_axis_name="c", subcore_axis_name="s")
_SC_PARAMS = pltpu.CompilerParams(use_tc_tiling_on_sc=False)


def _tile_start(wid):
    return NCHB * wid + jnp.maximum(wid - (NW - XTRA), 0)


@functools.partial(
    pl.kernel,
    out_type=jax.ShapeDtypeStruct((NC, NP, 16), jnp.float32),
    mesh=_MESH,
    scratch_types=[
        pltpu.VMEM((IDXR, 2, CHUNK), jnp.int32),
        pltpu.VMEM((CHUNK, 16), jnp.float32),
        pltpu.VMEM((ZROWS, 16), jnp.float32),
        pltpu.VMEM_SHARED((NP, 16), jnp.float32),
        pltpu.SemaphoreType.DMA,
    ],
    compiler_params=_SC_PARAMS,
)
def _deg(ei_hbm, out_hbm, idx_v, ones_v, z_v, acc, sem):
    c = lax.axis_index("c")
    s = lax.axis_index("s")
    wid = c * NS + s
    extra = wid >= NW - XTRA

    @pl.loop(0, CHUNK)
    def _(r):
        ones_v[r, :] = jnp.ones((16,), jnp.float32)

    @pl.loop(0, ZROWS)
    def _(r):
        z_v[r, :] = jnp.zeros((16,), jnp.float32)

    for k in range(RPT // ZROWS):
        pltpu.sync_copy(z_v, acc.at[pl.ds(s * RPT + k * ZROWS, ZROWS)])
    plsc.subcore_barrier()

    pltpu.sync_copy(ei_hbm.at[pl.ds(_tile_start(wid), IDXR)], idx_v)

    @pl.loop(0, NCHB)
    def _(j):
        pltpu.async_copy(ones_v, acc.at[idx_v.at[j, 1]], sem, add=True)

    @pl.when(extra)
    def _():
        pltpu.async_copy(ones_v, acc.at[idx_v.at[NCHB, 1]], sem, add=True)
        pltpu.make_async_copy(ones_v, acc.at[idx_v.at[NCHB, 1]], sem).wait()

    @pl.loop(0, NCHB)
    def _(j):
        pltpu.make_async_copy(ones_v, acc.at[idx_v.at[j, 1]], sem).wait()

    plsc.subcore_barrier()
    pltpu.sync_copy(acc.at[pl.ds(s * RPT, RPT)],
                    out_hbm.at[c, pl.ds(s * RPT, RPT)])


NBUF = 6
GAHEAD = 2
assert NCHB % NBUF == 0 and NCHB >= 3 * NBUF


@functools.partial(
    pl.kernel,
    out_type=jax.ShapeDtypeStruct((NC, 2, NP, DH), jnp.float32),
    mesh=_MESH,
    scratch_types=[
        pltpu.VMEM((IDXR, 2, CHUNK), jnp.int32),
    ] + [pltpu.VMEM((CHUNK, DH), jnp.float32) for _ in range(NBUF)]
      + [pltpu.VMEM((ZROWS, DH), jnp.float32),
         pltpu.VMEM_SHARED((NP, DH), jnp.float32)]
      + [pltpu.SemaphoreType.DMA for _ in range(2 * NBUF)],
    compiler_params=_SC_PARAMS,
)
def _edge_pass(y_hbm, ei_hbm, out_hbm,
               idx_v, b0, b1, b2, b3, b4, b5, z_v, acc,
               g0, g1, g2, g3, g4, g5, s0, s1, s2, s3, s4, s5):
    c = lax.axis_index("c")
    s = lax.axis_index("s")
    wid = c * NS + s
    extra = wid >= NW - XTRA
    bufs = (b0, b1, b2, b3, b4, b5)
    gsems = (g0, g1, g2, g3, g4, g5)
    ssems = (s0, s1, s2, s3, s4, s5)

    @pl.loop(0, ZROWS)
    def _(r):
        for cb in range(DH // 16):
            z_v[r, pl.ds(cb * 16, 16)] = jnp.zeros((16,), jnp.float32)

    pltpu.sync_copy(ei_hbm.at[pl.ds(_tile_start(wid), IDXR)], idx_v)

    @pl.loop(0, IDXR)
    def _(r):
        for cb in range(CHUNK // 16):
            sl = (r, 0, pl.ds(cb * 16, 16))
            idx_v[sl] = idx_v[sl] * 2

    for h in range(2):
        tab = y_hbm.at[pl.ds(h, 2 * N - h)]

        def gath(j, b):
            pltpu.async_copy(tab.at[idx_v.at[j, 0]], bufs[b], gsems[b])

        def gwait(j, b):
            pltpu.make_async_copy(tab.at[idx_v.at[j, 0]], bufs[b],
                                  gsems[b]).wait()

        def scat(j, b):
            pltpu.async_copy(bufs[b], acc.at[idx_v.at[j, 1]], ssems[b],
                             add=True)

        def swait(j, b):
            pltpu.make_async_copy(bufs[b], acc.at[idx_v.at[j, 1]],
                                  ssems[b]).wait()

        def step(j, b, do_swait, do_gath):
            gwait(j, b)
            scat(j, b)
            bn = (b + GAHEAD) % NBUF
            if do_swait:
                swait(j - (NBUF - GAHEAD), bn)
            if do_gath:
                gath(j + GAHEAD, bn)

        for k in range(RPT // ZROWS):
            pltpu.sync_copy(z_v, acc.at[pl.ds(s * RPT + k * ZROWS, ZROWS)])
        plsc.subcore_barrier()

        @pl.when(extra)
        def _():
            gath(NCHB, 0)
            gwait(NCHB, 0)
            scat(NCHB, 0)
            swait(NCHB, 0)

        for j in range(GAHEAD):
            gath(j, j)
        for j in range(NBUF):
            step(j, j, j >= NBUF - GAHEAD, True)

        @pl.loop(1, NCHB // NBUF - 1)
        def _(grp):
            j0 = grp * NBUF
            for k in range(NBUF):
                step(j0 + k, k, True, True)

        for j in range(NCHB - NBUF, NCHB):
            b = j % NBUF
            step(j, b, j < NCHB - GAHEAD, j < NCHB - GAHEAD)
        for j in range(NCHB - NBUF, NCHB):
            swait(j, j % NBUF)

        plsc.subcore_barrier()
        pltpu.sync_copy(acc.at[pl.ds(s * RPT, RPT)],
                        out_hbm.at[c, h, pl.ds(s * RPT, RPT)])


def _dinv_blk(deg_ref):
    d = deg_ref[...]
    return lax.rsqrt(1.0 + d[0, :, 0:1] + d[1, :, 0:1])


def _merge(o_ref, y_ref):
    o = o_ref[...]
    a = o[0, 0] + o[1, 0]
    b = o[0, 1] + o[1, 1]
    even = jnp.concatenate([a[:, :DH], b[:, :DH]], axis=1)
    odd = jnp.concatenate([a[:, DH:], b[:, DH:]], axis=1)
    t = jnp.stack([even, odd], axis=1).reshape(MBLK, D)
    return t + y_ref[...]


def _mm_scale_body(x_ref, w_ref, deg_ref, y_ref):
    xw = jnp.dot(x_ref[...], w_ref[...], preferred_element_type=jnp.float32)
    y_ref[...] = xw * _dinv_blk(deg_ref)


def _mm_scale(x, w, degp):
    return pl.pallas_call(
        _mm_scale_body,
        grid=(NBLK,),
        in_specs=[
            pl.BlockSpec((MBLK, D), lambda i: (i, 0)),
            pl.BlockSpec((D, D), lambda i: (0, 0)),
            pl.BlockSpec((NC, MBLK, 16), lambda i: (0, i, 0)),
        ],
        out_specs=pl.BlockSpec((MBLK, D), lambda i: (i, 0)),
        out_shape=jax.ShapeDtypeStruct((N, D), jnp.float32),
    )(x, w, degp)


def _pool_update(i, g_ref, bt_ref, z):
    @pl.when(i == 0)
    def _():
        g_ref[...] = jnp.zeros_like(g_ref)

    bb = bt_ref[0, 0, :]
    gi = lax.broadcasted_iota(jnp.int32, (G, MBLK), 0)
    m = (gi == bb[None, :]).astype(jnp.float32)
    g_ref[...] += jnp.dot(m, z, preferred_element_type=jnp.float32)


def _combine_mm_body(o_ref, y_ref, deg_ref, b_ref, a_ref, w_ref,
                     z_ref, y2_ref):
    dinv = _dinv_blk(deg_ref)
    u = _merge(o_ref, y_ref) * dinv + b_ref[...]
    z = jnp.where(u >= 0, u, a_ref[...] * u)
    z_ref[...] = z
    y2_ref[...] = jnp.dot(z, w_ref[...],
                          preferred_element_type=jnp.float32) * dinv


def _combine_mm(o, y, degp, b, a, w):
    return pl.pallas_call(
        _combine_mm_body,
        grid=(NBLK,),
        in_specs=[
            pl.BlockSpec((NC, 2, MBLK // 2, D), lambda i: (0, 0, i, 0)),
            pl.BlockSpec((MBLK, D), lambda i: (i, 0)),
            pl.BlockSpec((NC, MBLK, 16), lambda i: (0, i, 0)),
            pl.BlockSpec((1, D), lambda i: (0, 0)),
            pl.BlockSpec((1, D), lambda i: (0, 0)),
            pl.BlockSpec((D, D), lambda i: (0, 0)),
        ],
        out_specs=[
            pl.BlockSpec((MBLK, D), lambda i: (i, 0)),
            pl.BlockSpec((MBLK, D), lambda i: (i, 0)),
        ],
        out_shape=[
            jax.ShapeDtypeStruct((N, D), jnp.float32),
            jax.ShapeDtypeStruct((N, D), jnp.float32),
        ],
    )(o, y, degp, b, a, w)


def _pool_body(z_ref, bt_ref, g_ref):
    _pool_update(pl.program_id(0), g_ref, bt_ref, z_ref[...])


def _pool(z, batch3):
    return pl.pallas_call(
        _pool_body,
        grid=(NBLK,),
        in_specs=[
            pl.BlockSpec((MBLK, D), lambda i: (i, 0)),
            pl.BlockSpec((1, 1, MBLK), lambda i: (i, 0, 0)),
        ],
        out_specs=pl.BlockSpec((G, D), lambda i: (0, 0)),
        out_shape=jax.ShapeDtypeStruct((G, D), jnp.float32),
    )(z, batch3)


def _combine_pool_body(o_ref, y_ref, deg_ref, b_ref, a_ref, bt_ref, g1_ref,
                       z_ref, g_ref):
    i = pl.program_id(0)
    dinv = _dinv_blk(deg_ref)
    u = _merge(o_ref, y_ref) * dinv + b_ref[...]
    z = jnp.where(u >= 0, u, a_ref[...] * u)
    z_ref[...] = z

    @pl.when(i == 0)
    def _():
        g_ref[:, :D] = g1_ref[...]
        g_ref[:, D:] = jnp.zeros_like(g1_ref)

    bb = bt_ref[0, 0, :]
    gi = lax.broadcasted_iota(jnp.int32, (G, MBLK), 0)
    m = (gi == bb[None, :]).astype(jnp.float32)
    g_ref[:, D:] += jnp.dot(m, z, preferred_element_type=jnp.float32)


def _combine_pool(o, y, degp, b, a, batch3, g1):
    return pl.pallas_call(
        _combine_pool_body,
        grid=(NBLK,),
        in_specs=[
            pl.BlockSpec((NC, 2, MBLK // 2, D), lambda i: (0, 0, i, 0)),
            pl.BlockSpec((MBLK, D), lambda i: (i, 0)),
            pl.BlockSpec((NC, MBLK, 16), lambda i: (0, i, 0)),
            pl.BlockSpec((1, D), lambda i: (0, 0)),
            pl.BlockSpec((1, D), lambda i: (0, 0)),
            pl.BlockSpec((1, 1, MBLK), lambda i: (i, 0, 0)),
            pl.BlockSpec((G, D), lambda i: (0, 0)),
        ],
        out_specs=[
            pl.BlockSpec((MBLK, D), lambda i: (i, 0)),
            pl.BlockSpec((G, 2 * D), lambda i: (0, 0)),
        ],
        out_shape=[
            jax.ShapeDtypeStruct((N, D), jnp.float32),
            jax.ShapeDtypeStruct((G, 2 * D), jnp.float32),
        ],
    )(o, y, degp, b, a, batch3, g1)


@jax.jit
def kernel(x, edge_index, batch, W1, b1, a1, W2, b2, a2):
    ei = edge_index.reshape(2, ROWS, CHUNK).transpose(1, 0, 2)
    batch3 = batch.reshape(NBLK, 1, MBLK)
    b1r = b1.reshape(1, D)
    a1r = a1.reshape(1, D)
    b2r = b2.reshape(1, D)
    a2r = a2.reshape(1, D)

    degp = _deg(ei)
    y1 = _mm_scale(x, W1, degp)
    o1 = _edge_pass(y1.reshape(2 * N, DH), ei)
    o1p = o1.reshape(NC, 2, NP // 2, D)
    z1, y2 = _combine_mm(o1p, y1, degp, b1r, a1r, W2)
    o2 = _edge_pass(y2.reshape(2 * N, DH), ei)
    o2p = o2.reshape(NC, 2, NP // 2, D)
    g1 = _pool(z1, batch3)
    z2, g = _combine_pool(o2p, y2, degp, b2r, a2r, batch3, g1)
    return (z2, g)
